# baseline (speedup 1.0000x reference)
import functools

import jax
import jax.numpy as jnp
from jax import lax
from jax.experimental import pallas as pl
from jax.experimental.pallas import tpu as pltpu
from jax.experimental.pallas import tpu_sc as plsc

_NC = 2
_NS = 16
_NW = _NC * _NS
_CB = 4
_NBUF = 8


def _gather_body(table_hbm, idx_hbm, out_hbm, idx_v, rows_v, *sems):
    gsems, osems = sems[:_NBUF], sems[_NBUF:]
    wid = lax.axis_index("s") * _NC + lax.axis_index("c")
    fields = out_hbm.shape[1]
    rows_pw = idx_v.shape[0] // fields
    nchunk = rows_pw // _CB
    npc = _CB * fields
    b0 = wid * rows_pw
    pltpu.sync_copy(idx_hbm.at[pl.ds(b0 * fields, rows_pw * fields)], idx_v)

    for b in range(_NBUF):
        pltpu.async_copy(
            table_hbm.at[idx_v.at[pl.ds(b * npc, npc)]],
            rows_v.at[b],
            gsems[b],
        )

    @pl.loop(0, nchunk, step=_NBUF)
    def _outer(g):
        for b in range(_NBUF):
            j = g + b
            slot = b

            pltpu.make_async_copy(
                table_hbm.at[idx_v.at[pl.ds(0, npc)]],
                rows_v.at[slot],
                gsems[slot],
            ).wait()
            for r in range(_CB):
                pltpu.async_copy(
                    rows_v.at[slot, pl.ds(r * fields, fields)],
                    out_hbm.at[b0 + j * _CB + r],
                    osems[slot],
                )

            @pl.when(j + _NBUF < nchunk)
            def _():
                for r in range(_CB):
                    pltpu.make_async_copy(
                        rows_v.at[slot, pl.ds(r * fields, fields)],
                        out_hbm.at[b0],
                        osems[slot],
                    ).wait()
                pltpu.async_copy(
                    table_hbm.at[idx_v.at[pl.ds((j + _NBUF) * npc, npc)]],
                    rows_v.at[slot],
                    gsems[slot],
                )

    for b in range(_NBUF):
        for r in range(_CB):
            pltpu.make_async_copy(
                rows_v.at[b, pl.ds(r * fields, fields)], out_hbm.at[b0], osems[b]
            ).wait()


def kernel(x, weight):
    batch, fields = x.shape
    depth = weight.shape[1]
    total = batch * fields
    per_w = total // _NW
    idx = x.reshape(total)

    call = pl.kernel(
        _gather_body,
        out_type=jax.ShapeDtypeStruct((batch, fields, depth), jnp.float32),
        mesh=plsc.VectorSubcoreMesh(core_axis_name="c", subcore_axis_name="s"),
        scratch_types=[
            pltpu.VMEM((per_w,), jnp.int32),
            pltpu.VMEM((_NBUF, _CB * fields, depth), jnp.float32),
        ] + [pltpu.SemaphoreType.DMA] * (2 * _NBUF),
        compiler_params=pltpu.CompilerParams(use_tc_tiling_on_sc=False),
    )
    return call(weight, idx)


# speedup vs baseline: 1.0025x; 1.0025x over previous
import functools

import jax
import jax.numpy as jnp
from jax import lax
from jax.experimental import pallas as pl
from jax.experimental.pallas import tpu as pltpu
from jax.experimental.pallas import tpu_sc as plsc

_NC = 2
_NS = 16
_NW = _NC * _NS
_CH = 128
_NBUF = 13


def _gather_body(table_hbm, idx_hbm, out_hbm, idx_v, rows_v, *sems):
    gsems, osems = sems[:_NBUF], sems[_NBUF:]
    wid = lax.axis_index("s") * _NC + lax.axis_index("c")
    nchunk = idx_v.shape[0] // _CH
    base = wid * (nchunk * _CH)
    pltpu.sync_copy(idx_hbm.at[pl.ds(base, nchunk * _CH)], idx_v)

    for b in range(_NBUF):
        pltpu.async_copy(
            table_hbm.at[idx_v.at[pl.ds(b * _CH, _CH)]], rows_v.at[b], gsems[b]
        )

    @pl.loop(0, nchunk, step=_NBUF)
    def _outer(g):
        for b in range(_NBUF):
            j = g + b
            slot = b

            pltpu.make_async_copy(
                table_hbm.at[idx_v.at[pl.ds(0, _CH)]], rows_v.at[slot], gsems[slot]
            ).wait()
            pltpu.async_copy(
                rows_v.at[slot], out_hbm.at[pl.ds(base + j * _CH, _CH)], osems[slot]
            )

            @pl.when(j + _NBUF < nchunk)
            def _():
                pltpu.make_async_copy(
                    rows_v.at[slot], out_hbm.at[pl.ds(0, _CH)], osems[slot]
                ).wait()
                pltpu.async_copy(
                    table_hbm.at[idx_v.at[pl.ds((j + _NBUF) * _CH, _CH)]],
                    rows_v.at[slot],
                    gsems[slot],
                )

    for b in range(_NBUF):
        pltpu.make_async_copy(
            rows_v.at[b], out_hbm.at[pl.ds(0, _CH)], osems[b]
        ).wait()


def kernel(x, weight):
    batch, fields = x.shape
    depth = weight.shape[1]
    total = batch * fields
    per_w = total // _NW
    idx = x.reshape(total)

    call = pl.kernel(
        _gather_body,
        out_type=jax.ShapeDtypeStruct((total, depth), jnp.float32),
        mesh=plsc.VectorSubcoreMesh(core_axis_name="c", subcore_axis_name="s"),
        scratch_types=[
            pltpu.VMEM((per_w,), jnp.int32),
            pltpu.VMEM((_NBUF, _CH, depth), jnp.float32),
        ] + [pltpu.SemaphoreType.DMA] * (2 * _NBUF),
        compiler_params=pltpu.CompilerParams(use_tc_tiling_on_sc=False),
    )
    out = call(weight, idx)
    return out.reshape(batch, fields, depth)
